# baseline (speedup 1.0000x reference)
import functools

import jax
import jax.numpy as jnp
from jax import lax
from jax.experimental import pallas as pl
from jax.experimental.pallas import tpu as pltpu
from jax.experimental.pallas import tpu_sc as plsc

N = 10000
D = 128
NT = 16
NC = 2
NW = NC * NT
C = 128
NP = 10240
SLAB = NP // NT

_mesh = plsc.VectorSubcoreMesh(core_axis_name="c", subcore_axis_name="s")


def _make_hist(K, G):
  @functools.partial(
      pl.kernel,
      out_type=jax.ShapeDtypeStruct((NC, NP), jnp.float32),
      mesh=_mesh,
      scratch_types=[
          pltpu.VMEM((K, C), jnp.int32),
          pltpu.VMEM((C,), jnp.float32),
          pltpu.VMEM_SHARED((NP,), jnp.float32),
          pltpu.SemaphoreType.DMA,
      ],
  )
  def hist(dst_hbm, zeros_hbm, cnt_hbm, idx_v, ones_v, acc_sh, sem):
    cid = lax.axis_index("c")
    sid = lax.axis_index("s")
    w = cid * NT + sid
    for i in range(C // 16):
      ones_v[pl.ds(i * 16, 16)] = jnp.ones((16,), jnp.float32)
    pltpu.sync_copy(zeros_hbm, acc_sh.at[pl.ds(sid * SLAB, SLAB)])
    pltpu.sync_copy(dst_hbm.at[w], idx_v)
    plsc.subcore_barrier()

    def body(j, carry):
      pltpu.sync_copy(ones_v, acc_sh.at[idx_v.at[j]], add=True)
      return carry

    lax.fori_loop(0, K, body, 0)
    plsc.subcore_barrier()
    pltpu.sync_copy(acc_sh.at[pl.ds(sid * SLAB, SLAB)],
                    cnt_hbm.at[cid, pl.ds(sid * SLAB, SLAB)])

  return hist


def _make_agg(K, G):
  assert K % G == 0 and G % 2 == 0

  @functools.partial(
      pl.kernel,
      out_type=jax.ShapeDtypeStruct((NC, NP, D), jnp.float32),
      mesh=_mesh,
      scratch_types=[
          pltpu.VMEM((G, C), jnp.int32),
          pltpu.VMEM((G, C), jnp.int32),
          pltpu.VMEM((C, D), jnp.float32),
          pltpu.VMEM((C, D), jnp.float32),
          pltpu.VMEM_SHARED((NP, D), jnp.float32),
          pltpu.SemaphoreType.DMA,
          pltpu.SemaphoreType.DMA,
          pltpu.SemaphoreType.DMA,
          pltpu.SemaphoreType.DMA,
      ],
  )
  def agg(src_hbm, dst_hbm, g_hbm, zrow_hbm, out_hbm,
          srcv, dstv, buf0, buf1, acc_sh, sem0, sem1, ssem0, ssem1):
    cid = lax.axis_index("c")
    sid = lax.axis_index("s")
    w = cid * NT + sid
    for z in range(SLAB // 128):
      pltpu.sync_copy(zrow_hbm, acc_sh.at[pl.ds(sid * SLAB + z * 128, 128)])
    plsc.subcore_barrier()

    def start0(j):
      return pltpu.async_copy(g_hbm.at[srcv.at[j]], buf0, sem0)

    def start1(j):
      return pltpu.async_copy(g_hbm.at[srcv.at[j]], buf1, sem1)

    def wait0():
      pltpu.make_async_copy(g_hbm.at[srcv.at[0]], buf0, sem0).wait()

    def wait1():
      pltpu.make_async_copy(g_hbm.at[srcv.at[0]], buf1, sem1).wait()

    def sstart0(j):
      return pltpu.async_copy(buf0, acc_sh.at[dstv.at[j]], ssem0, add=True)

    def sstart1(j):
      return pltpu.async_copy(buf1, acc_sh.at[dstv.at[j]], ssem1, add=True)

    def swait0():
      pltpu.make_async_copy(buf0, acc_sh.at[dstv.at[0]], ssem0).wait()

    def swait1():
      pltpu.make_async_copy(buf1, acc_sh.at[dstv.at[0]], ssem1).wait()

    def phase(p, carry):
      pltpu.sync_copy(src_hbm.at[w].at[pl.ds(p * G, G)], srcv)
      pltpu.sync_copy(dst_hbm.at[w].at[pl.ds(p * G, G)], dstv)
      start0(0)
      start1(1)

      def body(t, c2):
        j0 = 2 * t
        wait0()
        sstart0(j0)
        wait1()
        sstart1(j0 + 1)
        swait0()
        start0(j0 + 2)
        swait1()
        start1(j0 + 3)
        return c2

      lax.fori_loop(0, G // 2 - 1, body, 0)
      wait0()
      sstart0(G - 2)
      wait1()
      sstart1(G - 1)
      swait0()
      swait1()
      return carry

    lax.fori_loop(0, K // G, phase, 0)
    plsc.subcore_barrier()
    pltpu.sync_copy(acc_sh.at[pl.ds(sid * SLAB, SLAB)],
                    out_hbm.at[cid, pl.ds(sid * SLAB, SLAB)])

  return agg


def _dinv_of(cnt_blk):
  return lax.rsqrt(jnp.maximum(cnt_blk + 1.0, 1.0))


def _mm_scale_body(cnt_ref, x_ref, w_ref, o_ref):
  h = jnp.dot(x_ref[...], w_ref[...], preferred_element_type=jnp.float32)
  o_ref[...] = h * _dinv_of(cnt_ref[...])


def _combine_mm_body(cnt_ref, p_ref, g_ref, b_ref, w_ref, o_ref):
  dinv = _dinv_of(cnt_ref[...])
  t = dinv * (p_ref[0] + p_ref[1] + g_ref[...]) + b_ref[...]
  t = jnp.maximum(t, 0.0)
  h = jnp.dot(t, w_ref[...], preferred_element_type=jnp.float32)
  o_ref[...] = h * dinv


def _combine_body(cnt_ref, p_ref, g_ref, b_ref, o_ref):
  dinv = _dinv_of(cnt_ref[...])
  o_ref[...] = dinv * (p_ref[0] + p_ref[1] + g_ref[...]) + b_ref[...]


_BR = 2000


def _tc_mm_scale(cnt_col, x, w):
  return pl.pallas_call(
      _mm_scale_body,
      grid=(N // _BR,),
      in_specs=[
          pl.BlockSpec((_BR, 1), lambda i: (i, 0)),
          pl.BlockSpec((_BR, D), lambda i: (i, 0)),
          pl.BlockSpec((D, D), lambda i: (0, 0)),
      ],
      out_specs=pl.BlockSpec((_BR, D), lambda i: (i, 0)),
      out_shape=jax.ShapeDtypeStruct((N, D), jnp.float32),
  )(cnt_col, x, w)


def _tc_combine_mm(cnt_col, p, g, b_row, w):
  return pl.pallas_call(
      _combine_mm_body,
      grid=(N // _BR,),
      in_specs=[
          pl.BlockSpec((_BR, 1), lambda i: (i, 0)),
          pl.BlockSpec((NC, _BR, D), lambda i: (0, i, 0)),
          pl.BlockSpec((_BR, D), lambda i: (i, 0)),
          pl.BlockSpec((1, D), lambda i: (0, 0)),
          pl.BlockSpec((D, D), lambda i: (0, 0)),
      ],
      out_specs=pl.BlockSpec((_BR, D), lambda i: (i, 0)),
      out_shape=jax.ShapeDtypeStruct((N, D), jnp.float32),
  )(cnt_col, p, g, b_row, w)


def _tc_combine(cnt_col, p, g, b_row):
  return pl.pallas_call(
      _combine_body,
      grid=(N // _BR,),
      in_specs=[
          pl.BlockSpec((_BR, 1), lambda i: (i, 0)),
          pl.BlockSpec((NC, _BR, D), lambda i: (0, i, 0)),
          pl.BlockSpec((_BR, D), lambda i: (i, 0)),
          pl.BlockSpec((1, D), lambda i: (0, 0)),
      ],
      out_specs=pl.BlockSpec((_BR, D), lambda i: (i, 0)),
      out_shape=jax.ShapeDtypeStruct((N, D), jnp.float32),
  )(cnt_col, p, g, b_row)


def kernel(x, edge_index, W1, b1, W2, b2):
  E = edge_index.shape[1]
  G = 40
  K = -(-E // (NW * C))
  K = -(-K // G) * G
  EP = NW * K * C

  src = edge_index[0].astype(jnp.int32)
  dst = edge_index[1].astype(jnp.int32)
  pad = EP - E
  pad_src = jnp.arange(pad, dtype=jnp.int32) % N
  pad_dst = N + (jnp.arange(pad, dtype=jnp.int32) % (NP - N))
  src_r = jnp.concatenate([src, pad_src]).reshape(NW, K, C)
  dst_r = jnp.concatenate([dst, pad_dst]).reshape(NW, K, C)

  zeros_slab = jnp.zeros((SLAB,), jnp.float32)
  zrow = jnp.zeros((128, D), jnp.float32)

  hist = _make_hist(K, G)
  agg = _make_agg(K, G)

  cnt = hist(dst_r, zeros_slab)
  cnt_col = (cnt[0, :N] + cnt[1, :N]).reshape(N, 1)

  g1 = _tc_mm_scale(cnt_col, x, W1)
  p1 = agg(src_r, dst_r, g1, zrow)
  g2 = _tc_combine_mm(cnt_col, p1, g1, b1.reshape(1, D), W2)
  p2 = agg(src_r, dst_r, g2, zrow)
  out = _tc_combine(cnt_col, p2, g2, b2.reshape(1, D))
  return out


# speedup vs baseline: 1.2165x; 1.2165x over previous
import functools

import jax
import jax.numpy as jnp
from jax import lax
from jax.experimental import pallas as pl
from jax.experimental.pallas import tpu as pltpu
from jax.experimental.pallas import tpu_sc as plsc

N = 10000
D = 128
NT = 16
NC = 2
NW = NC * NT
C = 128
NP = 10240
SLAB = NP // NT

_mesh = plsc.VectorSubcoreMesh(core_axis_name="c", subcore_axis_name="s")


def _make_hist(K, G):
  @functools.partial(
      pl.kernel,
      out_type=jax.ShapeDtypeStruct((NC, NP), jnp.float32),
      mesh=_mesh,
      scratch_types=[
          pltpu.VMEM((K, C), jnp.int32),
          pltpu.VMEM((C,), jnp.float32),
          pltpu.VMEM_SHARED((NP,), jnp.float32),
          pltpu.SemaphoreType.DMA,
      ],
  )
  def hist(dst_hbm, zeros_hbm, cnt_hbm, idx_v, ones_v, acc_sh, sem):
    cid = lax.axis_index("c")
    sid = lax.axis_index("s")
    w = cid * NT + sid
    for i in range(C // 16):
      ones_v[pl.ds(i * 16, 16)] = jnp.ones((16,), jnp.float32)
    pltpu.sync_copy(zeros_hbm, acc_sh.at[pl.ds(sid * SLAB, SLAB)])
    pltpu.sync_copy(dst_hbm.at[w], idx_v)
    plsc.subcore_barrier()

    def body(j, carry):
      pltpu.sync_copy(ones_v, acc_sh.at[idx_v.at[j]], add=True)
      return carry

    lax.fori_loop(0, K, body, 0)
    plsc.subcore_barrier()
    pltpu.sync_copy(acc_sh.at[pl.ds(sid * SLAB, SLAB)],
                    cnt_hbm.at[cid, pl.ds(sid * SLAB, SLAB)])

  return hist


def _make_agg(K, G):
  assert K % G == 0 and G % 2 == 0

  @functools.partial(
      pl.kernel,
      out_type=jax.ShapeDtypeStruct((NC, NP, D), jnp.float32),
      mesh=_mesh,
      scratch_types=[
          pltpu.VMEM((G, C), jnp.int32),
          pltpu.VMEM((G, C), jnp.int32),
          pltpu.VMEM((C, D), jnp.float32),
          pltpu.VMEM((C, D), jnp.float32),
          pltpu.VMEM_SHARED((NP, D), jnp.float32),
          pltpu.SemaphoreType.DMA,
          pltpu.SemaphoreType.DMA,
      ],
  )
  def agg(src_hbm, dst_hbm, g_hbm, zrow_hbm, out_hbm,
          srcv, dstv, buf0, buf1, acc_sh, sem0, sem1):
    cid = lax.axis_index("c")
    sid = lax.axis_index("s")
    w = cid * NT + sid
    for z in range(SLAB // 128):
      pltpu.sync_copy(zrow_hbm, acc_sh.at[pl.ds(sid * SLAB + z * 128, 128)])
    plsc.subcore_barrier()

    def start0(j):
      return pltpu.async_copy(g_hbm.at[srcv.at[j]], buf0, sem0)

    def start1(j):
      return pltpu.async_copy(g_hbm.at[srcv.at[j]], buf1, sem1)

    def wait0():
      pltpu.make_async_copy(g_hbm.at[srcv.at[0]], buf0, sem0).wait()

    def wait1():
      pltpu.make_async_copy(g_hbm.at[srcv.at[0]], buf1, sem1).wait()

    def phase(p, carry):
      pltpu.sync_copy(src_hbm.at[w].at[pl.ds(p * G, G)], srcv)
      pltpu.sync_copy(dst_hbm.at[w].at[pl.ds(p * G, G)], dstv)
      start0(0)

      def body(t, c2):
        j0 = 2 * t
        start1(j0 + 1)
        wait0()
        pltpu.sync_copy(buf0, acc_sh.at[dstv.at[j0]], add=True)
        start0(j0 + 2)
        wait1()
        pltpu.sync_copy(buf1, acc_sh.at[dstv.at[j0 + 1]], add=True)
        return c2

      lax.fori_loop(0, G // 2 - 1, body, 0)
      start1(G - 1)
      wait0()
      pltpu.sync_copy(buf0, acc_sh.at[dstv.at[G - 2]], add=True)
      wait1()
      pltpu.sync_copy(buf1, acc_sh.at[dstv.at[G - 1]], add=True)
      return carry

    lax.fori_loop(0, K // G, phase, 0)
    plsc.subcore_barrier()
    pltpu.sync_copy(acc_sh.at[pl.ds(sid * SLAB, SLAB)],
                    out_hbm.at[cid, pl.ds(sid * SLAB, SLAB)])

  return agg


def _dinv_of(cnt_blk):
  return lax.rsqrt(jnp.maximum(cnt_blk + 1.0, 1.0))


def _mm_body(x_ref, w_ref, o_ref):
  o_ref[...] = jnp.dot(x_ref[...], w_ref[...],
                       preferred_element_type=jnp.float32)


def _scale_body(cnt_ref, h_ref, o_ref):
  o_ref[...] = h_ref[...] * _dinv_of(cnt_ref[...])


def _combine_mm_body(cnt_ref, p_ref, g_ref, b_ref, w_ref, o_ref):
  dinv = _dinv_of(cnt_ref[...])
  t = dinv * (p_ref[0] + p_ref[1] + g_ref[...]) + b_ref[...]
  t = jnp.maximum(t, 0.0)
  h = jnp.dot(t, w_ref[...], preferred_element_type=jnp.float32)
  o_ref[...] = h * dinv


def _combine_body(cnt_ref, p_ref, g_ref, b_ref, o_ref):
  dinv = _dinv_of(cnt_ref[...])
  o_ref[...] = dinv * (p_ref[0] + p_ref[1] + g_ref[...]) + b_ref[...]


_BR = 2000


def _tc_mm(x, w):
  return pl.pallas_call(
      _mm_body,
      grid=(N // _BR,),
      in_specs=[
          pl.BlockSpec((_BR, D), lambda i: (i, 0)),
          pl.BlockSpec((D, D), lambda i: (0, 0)),
      ],
      out_specs=pl.BlockSpec((_BR, D), lambda i: (i, 0)),
      out_shape=jax.ShapeDtypeStruct((N, D), jnp.float32),
  )(x, w)


def _tc_scale(cnt_col, h):
  return pl.pallas_call(
      _scale_body,
      grid=(N // _BR,),
      in_specs=[
          pl.BlockSpec((_BR, 1), lambda i: (i, 0)),
          pl.BlockSpec((_BR, D), lambda i: (i, 0)),
      ],
      out_specs=pl.BlockSpec((_BR, D), lambda i: (i, 0)),
      out_shape=jax.ShapeDtypeStruct((N, D), jnp.float32),
  )(cnt_col, h)


def _tc_combine_mm(cnt_col, p, g, b_row, w):
  return pl.pallas_call(
      _combine_mm_body,
      grid=(N // _BR,),
      in_specs=[
          pl.BlockSpec((_BR, 1), lambda i: (i, 0)),
          pl.BlockSpec((NC, _BR, D), lambda i: (0, i, 0)),
          pl.BlockSpec((_BR, D), lambda i: (i, 0)),
          pl.BlockSpec((1, D), lambda i: (0, 0)),
          pl.BlockSpec((D, D), lambda i: (0, 0)),
      ],
      out_specs=pl.BlockSpec((_BR, D), lambda i: (i, 0)),
      out_shape=jax.ShapeDtypeStruct((N, D), jnp.float32),
  )(cnt_col, p, g, b_row, w)


def _tc_combine(cnt_col, p, g, b_row):
  return pl.pallas_call(
      _combine_body,
      grid=(N // _BR,),
      in_specs=[
          pl.BlockSpec((_BR, 1), lambda i: (i, 0)),
          pl.BlockSpec((NC, _BR, D), lambda i: (0, i, 0)),
          pl.BlockSpec((_BR, D), lambda i: (i, 0)),
          pl.BlockSpec((1, D), lambda i: (0, 0)),
      ],
      out_specs=pl.BlockSpec((_BR, D), lambda i: (i, 0)),
      out_shape=jax.ShapeDtypeStruct((N, D), jnp.float32),
  )(cnt_col, p, g, b_row)


def kernel(x, edge_index, W1, b1, W2, b2):
  E = edge_index.shape[1]
  G = 40
  K = -(-E // (NW * C))
  K = -(-K // G) * G
  EP = NW * K * C

  src = edge_index[0].astype(jnp.int32)
  dst = edge_index[1].astype(jnp.int32)
  pad = EP - E
  pad_src = jnp.arange(pad, dtype=jnp.int32) % N
  pad_dst = N + (jnp.arange(pad, dtype=jnp.int32) % (NP - N))
  src_r = jnp.concatenate([src, pad_src]).reshape(NW, K, C)
  dst_r = jnp.concatenate([dst, pad_dst]).reshape(NW, K, C)

  zeros_slab = jnp.zeros((SLAB,), jnp.float32)
  zrow = jnp.zeros((128, D), jnp.float32)

  hist = _make_hist(K, G)
  agg = _make_agg(K, G)

  cnt = hist(dst_r, zeros_slab)
  h1 = _tc_mm(x, W1)
  cnt_col = (cnt[0, :N] + cnt[1, :N]).reshape(N, 1)

  g1 = _tc_scale(cnt_col, h1)
  p1 = agg(src_r, dst_r, g1, zrow)
  g2 = _tc_combine_mm(cnt_col, p1, g1, b1.reshape(1, D), W2)
  p2 = agg(src_r, dst_r, g2, zrow)
  out = _tc_combine(cnt_col, p2, g2, b2.reshape(1, D))
  return out


# speedup vs baseline: 1.2543x; 1.0311x over previous
import functools

import jax
import jax.numpy as jnp
from jax import lax
from jax.experimental import pallas as pl
from jax.experimental.pallas import tpu as pltpu
from jax.experimental.pallas import tpu_sc as plsc

N = 10000
D = 128
NT = 16
NC = 2
NW = NC * NT
NP = 10240
SLAB = NP // NT

_mesh = plsc.VectorSubcoreMesh(core_axis_name="c", subcore_axis_name="s")


def _make_hist(K, G):
  @functools.partial(
      pl.kernel,
      out_type=jax.ShapeDtypeStruct((NC, NP), jnp.float32),
      mesh=_mesh,
      scratch_types=[
          pltpu.VMEM((K, CA), jnp.int32),
          pltpu.VMEM((CA,), jnp.float32),
          pltpu.VMEM_SHARED((NP,), jnp.float32),
          pltpu.SemaphoreType.DMA,
      ],
  )
  def hist(dst_hbm, zeros_hbm, cnt_hbm, idx_v, ones_v, acc_sh, sem):
    cid = lax.axis_index("c")
    sid = lax.axis_index("s")
    w = cid * NT + sid
    for i in range(CA // 16):
      ones_v[pl.ds(i * 16, 16)] = jnp.ones((16,), jnp.float32)
    pltpu.sync_copy(zeros_hbm, acc_sh.at[pl.ds(sid * SLAB, SLAB)])
    pltpu.sync_copy(dst_hbm.at[w], idx_v)
    plsc.subcore_barrier()

    def body(j, carry):
      pltpu.sync_copy(ones_v, acc_sh.at[idx_v.at[j]], add=True)
      return carry

    lax.fori_loop(0, K, body, 0)
    plsc.subcore_barrier()
    pltpu.sync_copy(acc_sh.at[pl.ds(sid * SLAB, SLAB)],
                    cnt_hbm.at[cid, pl.ds(sid * SLAB, SLAB)])

  return hist


CA = 64
NB = 4


def _make_agg(K, G):
  assert K % G == 0 and G % NB == 0

  @functools.partial(
      pl.kernel,
      out_type=jax.ShapeDtypeStruct((NC, NP, D), jnp.float32),
      mesh=_mesh,
      scratch_types=[
          pltpu.VMEM((G, CA), jnp.int32),
          pltpu.VMEM((G, CA), jnp.int32),
          pltpu.VMEM((NB, CA, D), jnp.float32),
          pltpu.VMEM_SHARED((NP, D), jnp.float32),
          [pltpu.SemaphoreType.DMA] * NB,
      ],
  )
  def agg(src_hbm, dst_hbm, g_hbm, zrow_hbm, out_hbm,
          srcv, dstv, bufs, acc_sh, sems):
    cid = lax.axis_index("c")
    sid = lax.axis_index("s")
    w = cid * NT + sid
    for z in range(SLAB // 128):
      pltpu.sync_copy(zrow_hbm, acc_sh.at[pl.ds(sid * SLAB + z * 128, 128)])
    plsc.subcore_barrier()

    def start(b, j):
      return pltpu.async_copy(g_hbm.at[srcv.at[j]], bufs.at[b], sems[b])

    def wait(b):
      pltpu.make_async_copy(g_hbm.at[srcv.at[0]], bufs.at[b], sems[b]).wait()

    def scat(b, j):
      pltpu.sync_copy(bufs.at[b], acc_sh.at[dstv.at[j]], add=True)

    def phase(p, carry):
      pltpu.sync_copy(src_hbm.at[w].at[pl.ds(p * G, G)], srcv)
      pltpu.sync_copy(dst_hbm.at[w].at[pl.ds(p * G, G)], dstv)
      for b in range(NB):
        start(b, b)

      def body(t, c2):
        j0 = NB * t
        for b in range(NB):
          wait(b)
          scat(b, j0 + b)
          start(b, j0 + b + NB)
        return c2

      lax.fori_loop(0, G // NB - 1, body, 0)
      for b in range(NB):
        wait(b)
        scat(b, G - NB + b)
      return carry

    lax.fori_loop(0, K // G, phase, 0)
    plsc.subcore_barrier()
    pltpu.sync_copy(acc_sh.at[pl.ds(sid * SLAB, SLAB)],
                    out_hbm.at[cid, pl.ds(sid * SLAB, SLAB)])

  return agg


def _dinv_of(cnt_blk):
  return lax.rsqrt(jnp.maximum(cnt_blk + 1.0, 1.0))


def _mm_scale_body(cnt_ref, x_ref, w_ref, o_ref):
  h = jnp.dot(x_ref[...], w_ref[...], preferred_element_type=jnp.float32)
  o_ref[...] = h * _dinv_of(cnt_ref[...])


def _combine_mm_body(cnt_ref, p_ref, g_ref, b_ref, w_ref, o_ref):
  dinv = _dinv_of(cnt_ref[...])
  t = dinv * (p_ref[0] + p_ref[1] + g_ref[...]) + b_ref[...]
  t = jnp.maximum(t, 0.0)
  h = jnp.dot(t, w_ref[...], preferred_element_type=jnp.float32)
  o_ref[...] = h * dinv


def _combine_body(cnt_ref, p_ref, g_ref, b_ref, o_ref):
  dinv = _dinv_of(cnt_ref[...])
  o_ref[...] = dinv * (p_ref[0] + p_ref[1] + g_ref[...]) + b_ref[...]


_BR = 2000


def _tc_mm_scale(cnt_col, x, w):
  return pl.pallas_call(
      _mm_scale_body,
      grid=(N // _BR,),
      in_specs=[
          pl.BlockSpec((_BR, 1), lambda i: (i, 0)),
          pl.BlockSpec((_BR, D), lambda i: (i, 0)),
          pl.BlockSpec((D, D), lambda i: (0, 0)),
      ],
      out_specs=pl.BlockSpec((_BR, D), lambda i: (i, 0)),
      out_shape=jax.ShapeDtypeStruct((N, D), jnp.float32),
  )(cnt_col, x, w)


def _tc_combine_mm(cnt_col, p, g, b_row, w):
  return pl.pallas_call(
      _combine_mm_body,
      grid=(N // _BR,),
      in_specs=[
          pl.BlockSpec((_BR, 1), lambda i: (i, 0)),
          pl.BlockSpec((NC, _BR, D), lambda i: (0, i, 0)),
          pl.BlockSpec((_BR, D), lambda i: (i, 0)),
          pl.BlockSpec((1, D), lambda i: (0, 0)),
          pl.BlockSpec((D, D), lambda i: (0, 0)),
      ],
      out_specs=pl.BlockSpec((_BR, D), lambda i: (i, 0)),
      out_shape=jax.ShapeDtypeStruct((N, D), jnp.float32),
  )(cnt_col, p, g, b_row, w)


def _tc_combine(cnt_col, p, g, b_row):
  return pl.pallas_call(
      _combine_body,
      grid=(N // _BR,),
      in_specs=[
          pl.BlockSpec((_BR, 1), lambda i: (i, 0)),
          pl.BlockSpec((NC, _BR, D), lambda i: (0, i, 0)),
          pl.BlockSpec((_BR, D), lambda i: (i, 0)),
          pl.BlockSpec((1, D), lambda i: (0, 0)),
      ],
      out_specs=pl.BlockSpec((_BR, D), lambda i: (i, 0)),
      out_shape=jax.ShapeDtypeStruct((N, D), jnp.float32),
  )(cnt_col, p, g, b_row)


def kernel(x, edge_index, W1, b1, W2, b2):
  E = edge_index.shape[1]
  G = 40
  K = -(-E // (NW * CA))
  K = -(-K // G) * G
  EP = NW * K * CA

  src = edge_index[0].astype(jnp.int32)
  dst = edge_index[1].astype(jnp.int32)
  pad = EP - E
  pad_src = jnp.arange(pad, dtype=jnp.int32) % N
  pad_dst = N + (jnp.arange(pad, dtype=jnp.int32) % (NP - N))
  src_r = jnp.concatenate([src, pad_src]).reshape(NW, K, CA)
  dst_r = jnp.concatenate([dst, pad_dst]).reshape(NW, K, CA)

  zeros_slab = jnp.zeros((SLAB,), jnp.float32)
  zrow = jnp.zeros((128, D), jnp.float32)

  hist = _make_hist(K, G)
  agg = _make_agg(K, G)

  cnt = hist(dst_r, zeros_slab)
  cnt_col = (cnt[0, :N] + cnt[1, :N]).reshape(N, 1)

  g1 = _tc_mm_scale(cnt_col, x, W1)
  p1 = agg(src_r, dst_r, g1, zrow)
  g2 = _tc_combine_mm(cnt_col, p1, g1, b1.reshape(1, D), W2)
  p2 = agg(src_r, dst_r, g2, zrow)
  out = _tc_combine(cnt_col, p2, g2, b2.reshape(1, D))
  return out


# speedup vs baseline: 1.2870x; 1.0261x over previous
import functools

import jax
import jax.numpy as jnp
from jax import lax
from jax.experimental import pallas as pl
from jax.experimental.pallas import tpu as pltpu
from jax.experimental.pallas import tpu_sc as plsc

N = 10000
D = 128
NT = 16
NC = 2
NW = NC * NT
NP = 10240
SLAB = NP // NT

_mesh = plsc.VectorSubcoreMesh(core_axis_name="c", subcore_axis_name="s")


CH = 128


def _make_hist(K, G):
  @functools.partial(
      pl.kernel,
      out_type=jax.ShapeDtypeStruct((NC, NP), jnp.float32),
      mesh=_mesh,
      scratch_types=[
          pltpu.VMEM((K, CH), jnp.int32),
          pltpu.VMEM((CH,), jnp.float32),
          pltpu.VMEM_SHARED((NP,), jnp.float32),
          pltpu.SemaphoreType.DMA,
      ],
  )
  def hist(dst_hbm, zeros_hbm, cnt_hbm, idx_v, ones_v, acc_sh, sem):
    cid = lax.axis_index("c")
    sid = lax.axis_index("s")
    w = cid * NT + sid
    for i in range(CH // 16):
      ones_v[pl.ds(i * 16, 16)] = jnp.ones((16,), jnp.float32)
    pltpu.sync_copy(zeros_hbm, acc_sh.at[pl.ds(sid * SLAB, SLAB)])
    pltpu.sync_copy(dst_hbm.at[w], idx_v)
    plsc.subcore_barrier()

    def body(j, carry):
      pltpu.sync_copy(ones_v, acc_sh.at[idx_v.at[j]], add=True)
      return carry

    lax.fori_loop(0, K, body, 0)
    plsc.subcore_barrier()
    pltpu.sync_copy(acc_sh.at[pl.ds(sid * SLAB, SLAB)],
                    cnt_hbm.at[cid, pl.ds(sid * SLAB, SLAB)])

  return hist


CA = 64
NB = 4


def _make_agg(K, G):
  assert K % G == 0 and G % NB == 0

  @functools.partial(
      pl.kernel,
      out_type=jax.ShapeDtypeStruct((NC, NP, D), jnp.float32),
      mesh=_mesh,
      scratch_types=[
          pltpu.VMEM((G, CA), jnp.int32),
          pltpu.VMEM((G, CA), jnp.int32),
          pltpu.VMEM((NB, CA, D), jnp.float32),
          pltpu.VMEM_SHARED((NP, D), jnp.float32),
          [pltpu.SemaphoreType.DMA] * NB,
      ],
  )
  def agg(src_hbm, dst_hbm, g_hbm, zrow_hbm, out_hbm,
          srcv, dstv, bufs, acc_sh, sems):
    cid = lax.axis_index("c")
    sid = lax.axis_index("s")
    w = cid * NT + sid
    for z in range(SLAB // 128):
      pltpu.sync_copy(zrow_hbm, acc_sh.at[pl.ds(sid * SLAB + z * 128, 128)])
    plsc.subcore_barrier()

    def start(b, j):
      return pltpu.async_copy(g_hbm.at[srcv.at[j]], bufs.at[b], sems[b])

    def wait(b):
      pltpu.make_async_copy(g_hbm.at[srcv.at[0]], bufs.at[b], sems[b]).wait()

    def scat(b, j):
      pltpu.sync_copy(bufs.at[b], acc_sh.at[dstv.at[j]], add=True)

    def phase(p, carry):
      pltpu.sync_copy(src_hbm.at[w].at[pl.ds(p * G, G)], srcv)
      pltpu.sync_copy(dst_hbm.at[w].at[pl.ds(p * G, G)], dstv)
      for b in range(NB):
        start(b, b)

      def body(t, c2):
        j0 = NB * t
        for b in range(NB):
          wait(b)
          scat(b, j0 + b)
          start(b, j0 + b + NB)
        return c2

      lax.fori_loop(0, G // NB - 1, body, 0)
      for b in range(NB):
        wait(b)
        scat(b, G - NB + b)
      return carry

    lax.fori_loop(0, K // G, phase, 0)
    plsc.subcore_barrier()
    pltpu.sync_copy(acc_sh.at[pl.ds(sid * SLAB, SLAB)],
                    out_hbm.at[cid, pl.ds(sid * SLAB, SLAB)])

  return agg


def _dinv_of(cnt_blk):
  return lax.rsqrt(jnp.maximum(cnt_blk + 1.0, 1.0))


def _mm_scale_body(cnt_ref, x_ref, w_ref, o_ref):
  h = jnp.dot(x_ref[...], w_ref[...], preferred_element_type=jnp.float32)
  o_ref[...] = h * _dinv_of(cnt_ref[...])


def _combine_mm_body(cnt_ref, p_ref, g_ref, b_ref, w_ref, o_ref):
  dinv = _dinv_of(cnt_ref[...])
  t = dinv * (p_ref[0] + p_ref[1] + g_ref[...]) + b_ref[...]
  t = jnp.maximum(t, 0.0)
  h = jnp.dot(t, w_ref[...], preferred_element_type=jnp.float32)
  o_ref[...] = h * dinv


def _combine_body(cnt_ref, p_ref, g_ref, b_ref, o_ref):
  dinv = _dinv_of(cnt_ref[...])
  o_ref[...] = dinv * (p_ref[0] + p_ref[1] + g_ref[...]) + b_ref[...]


_BR = 2000


def _tc_mm_scale(cnt_col, x, w):
  return pl.pallas_call(
      _mm_scale_body,
      grid=(N // _BR,),
      in_specs=[
          pl.BlockSpec((_BR, 1), lambda i: (i, 0)),
          pl.BlockSpec((_BR, D), lambda i: (i, 0)),
          pl.BlockSpec((D, D), lambda i: (0, 0)),
      ],
      out_specs=pl.BlockSpec((_BR, D), lambda i: (i, 0)),
      out_shape=jax.ShapeDtypeStruct((N, D), jnp.float32),
  )(cnt_col, x, w)


def _tc_combine_mm(cnt_col, p, g, b_row, w):
  return pl.pallas_call(
      _combine_mm_body,
      grid=(N // _BR,),
      in_specs=[
          pl.BlockSpec((_BR, 1), lambda i: (i, 0)),
          pl.BlockSpec((NC, _BR, D), lambda i: (0, i, 0)),
          pl.BlockSpec((_BR, D), lambda i: (i, 0)),
          pl.BlockSpec((1, D), lambda i: (0, 0)),
          pl.BlockSpec((D, D), lambda i: (0, 0)),
      ],
      out_specs=pl.BlockSpec((_BR, D), lambda i: (i, 0)),
      out_shape=jax.ShapeDtypeStruct((N, D), jnp.float32),
  )(cnt_col, p, g, b_row, w)


def _tc_combine(cnt_col, p, g, b_row):
  return pl.pallas_call(
      _combine_body,
      grid=(N // _BR,),
      in_specs=[
          pl.BlockSpec((_BR, 1), lambda i: (i, 0)),
          pl.BlockSpec((NC, _BR, D), lambda i: (0, i, 0)),
          pl.BlockSpec((_BR, D), lambda i: (i, 0)),
          pl.BlockSpec((1, D), lambda i: (0, 0)),
      ],
      out_specs=pl.BlockSpec((_BR, D), lambda i: (i, 0)),
      out_shape=jax.ShapeDtypeStruct((N, D), jnp.float32),
  )(cnt_col, p, g, b_row)


def kernel(x, edge_index, W1, b1, W2, b2):
  E = edge_index.shape[1]
  G = 40
  K = -(-E // (NW * CA))
  K = -(-K // G) * G
  EP = NW * K * CA

  src = edge_index[0].astype(jnp.int32)
  dst = edge_index[1].astype(jnp.int32)
  pad = EP - E
  pad_src = jnp.arange(pad, dtype=jnp.int32) % N
  pad_dst = N + (jnp.arange(pad, dtype=jnp.int32) % (NP - N))
  src_r = jnp.concatenate([src, pad_src]).reshape(NW, K, CA)
  dst_r = jnp.concatenate([dst, pad_dst]).reshape(NW, K, CA)

  zeros_slab = jnp.zeros((SLAB,), jnp.float32)
  zrow = jnp.zeros((128, D), jnp.float32)

  KH = K * CA // CH
  dst_h = dst_r.reshape(NW, KH, CH)
  hist = _make_hist(KH, G)
  agg = _make_agg(K, G)

  cnt = hist(dst_h, zeros_slab)
  cnt_col = (cnt[0, :N] + cnt[1, :N]).reshape(N, 1)

  g1 = _tc_mm_scale(cnt_col, x, W1)
  p1 = agg(src_r, dst_r, g1, zrow)
  g2 = _tc_combine_mm(cnt_col, p1, g1, b1.reshape(1, D), W2)
  p2 = agg(src_r, dst_r, g2, zrow)
  out = _tc_combine(cnt_col, p2, g2, b2.reshape(1, D))
  return out
